# Initial kernel scaffold; baseline (speedup 1.0000x reference)
import jax
import jax.numpy as jnp
from jax.experimental import pallas as pl


def kernel(x, edge_index, weight, bias):
    raise NotImplementedError("write your pallas kernel here")


# speedup vs baseline: 18.1585x; 18.1585x over previous
import functools

import jax
import jax.numpy as jnp
from jax import lax
from jax.experimental import pallas as pl
from jax.experimental.pallas import tpu as pltpu
from jax.experimental.pallas import tpu_sc as plsc

N_NODES = 10000
N_EDGES = 320000
F = 128

NC = 2
NS = 16
NW = NC * NS
B = 128

E_PAD = ((N_EDGES + NW * B - 1) // (NW * B)) * (NW * B)
NB = E_PAD // (NW * B)
N_ACC = ((N_NODES + 1 + NS * 16 - 1) // (NS * 16)) * (NS * 16)
TROWS = N_ACC // NS

_mesh = plsc.VectorSubcoreMesh(
    core_axis_name="c", subcore_axis_name="s", num_cores=NC, num_subcores=NS
)


@functools.partial(
    pl.kernel,
    out_type=(
        jax.ShapeDtypeStruct((NC, N_ACC), jnp.float32),
        jax.ShapeDtypeStruct((NW, NB, B), jnp.int32),
    ),
    mesh=_mesh,
    scratch_types=[
        pltpu.VMEM((NB, B), jnp.int32),
        pltpu.VMEM((NB, B), jnp.int32),
        pltpu.VMEM((B,), jnp.float32),
        pltpu.VMEM((TROWS,), jnp.float32),
        pltpu.VMEM_SHARED((N_ACC,), jnp.float32),
    ],
)
def _sc_deg(row_hbm, col_hbm, deg_hbm, dest_hbm, row_v, dest_v, ones_v, zv, deg_sh):
    c = lax.axis_index("c")
    s = lax.axis_index("s")
    w = c * NS + s

    pltpu.sync_copy(row_hbm.at[w], row_v)
    pltpu.sync_copy(col_hbm.at[w], dest_v)

    one16 = jnp.ones((16,), jnp.float32)
    zero16 = jnp.zeros((16,), jnp.float32)
    for g in range(B // 16):
        ones_v[pl.ds(g * 16, 16)] = one16

    def zfill(k, carry):
        zv[pl.ds(k * 16, 16)] = zero16
        return carry

    lax.fori_loop(0, TROWS // 16, zfill, 0)
    pltpu.sync_copy(zv, deg_sh.at[pl.ds(s * TROWS, TROWS)])

    dummy = jnp.full((16,), N_NODES, jnp.int32)

    def mask_body(j, carry):
        for g in range(B // 16):
            r = row_v[j, pl.ds(g * 16, 16)]
            cc = dest_v[j, pl.ds(g * 16, 16)]
            dest_v[j, pl.ds(g * 16, 16)] = jnp.where(r == cc, dummy, r)
        return carry

    lax.fori_loop(0, NB, mask_body, 0)
    pltpu.sync_copy(dest_v, dest_hbm.at[w])

    plsc.subcore_barrier()

    def add_body(j, carry):
        pltpu.sync_copy(ones_v, deg_sh.at[dest_v.at[j]], add=True)
        return carry

    lax.fori_loop(0, NB, add_body, 0)

    plsc.subcore_barrier()
    pltpu.sync_copy(deg_sh.at[pl.ds(s * TROWS, TROWS)], deg_hbm.at[c, pl.ds(s * TROWS, TROWS)])


@functools.partial(
    pl.kernel,
    out_type=jax.ShapeDtypeStruct((NC, N_ACC, F), jnp.float32),
    mesh=_mesh,
    scratch_types=[
        pltpu.VMEM((NB, B), jnp.int32),
        pltpu.VMEM((NB, B), jnp.int32),
        pltpu.VMEM((B, F), jnp.float32),
        pltpu.VMEM_SHARED((N_ACC, F), jnp.float32),
        pltpu.SemaphoreType.DMA,
    ],
)
def _sc_spmm(h2_hbm, col_hbm, dest_hbm, zinit_hbm, acc_hbm, col_v, dest_v, gbuf, acc_sh, sem):
    c = lax.axis_index("c")
    s = lax.axis_index("s")
    w = c * NS + s

    pltpu.sync_copy(col_hbm.at[w], col_v)
    pltpu.sync_copy(dest_hbm.at[w], dest_v)
    pltpu.sync_copy(zinit_hbm, acc_sh.at[pl.ds(s * TROWS, TROWS)])
    plsc.subcore_barrier()

    def body(j, carry):
        pltpu.async_copy(h2_hbm.at[col_v.at[j]], gbuf, sem).wait()
        pltpu.sync_copy(gbuf, acc_sh.at[dest_v.at[j]], add=True)
        return carry

    lax.fori_loop(0, NB, body, 0)

    plsc.subcore_barrier()
    pltpu.sync_copy(acc_sh.at[pl.ds(s * TROWS, TROWS)], acc_hbm.at[c, pl.ds(s * TROWS, TROWS)])


_RB = 1000


def _tc_h2_body(x_ref, w_ref, deg_ref, h2_ref):
    h = jnp.dot(x_ref[...], w_ref[...], preferred_element_type=jnp.float32)
    deg = deg_ref[:, 0] + deg_ref[:, 1] + 1.0
    dis = lax.rsqrt(deg)
    h2_ref[...] = h * dis[:, None]


_tc_h2 = pl.pallas_call(
    _tc_h2_body,
    grid=(N_NODES // _RB,),
    in_specs=[
        pl.BlockSpec((_RB, F), lambda i: (i, 0)),
        pl.BlockSpec((F, F), lambda i: (0, 0)),
        pl.BlockSpec((_RB, NC), lambda i: (i, 0)),
    ],
    out_specs=pl.BlockSpec((_RB, F), lambda i: (i, 0)),
    out_shape=jax.ShapeDtypeStruct((N_NODES, F), jnp.float32),
)


def _tc_out_body(acc_ref, h2_ref, deg_ref, b_ref, o_ref):
    acc = acc_ref[0] + acc_ref[1]
    deg = deg_ref[:, 0] + deg_ref[:, 1] + 1.0
    dis = lax.rsqrt(deg)
    o_ref[...] = (acc + h2_ref[...]) * dis[:, None] + b_ref[...]


_tc_out = pl.pallas_call(
    _tc_out_body,
    grid=(N_NODES // _RB,),
    in_specs=[
        pl.BlockSpec((NC, _RB, F), lambda i: (0, i, 0)),
        pl.BlockSpec((_RB, F), lambda i: (i, 0)),
        pl.BlockSpec((_RB, NC), lambda i: (i, 0)),
        pl.BlockSpec((1, F), lambda i: (0, 0)),
    ],
    out_specs=pl.BlockSpec((_RB, F), lambda i: (i, 0)),
    out_shape=jax.ShapeDtypeStruct((N_NODES, F), jnp.float32),
)


def kernel(x, edge_index, weight, bias):
    assert x.shape == (N_NODES, F) and edge_index.shape == (2, N_EDGES)
    row = edge_index[0]
    col = edge_index[1]
    pad = E_PAD - N_EDGES
    zpad = jnp.zeros((pad,), jnp.int32)
    row_p = jnp.concatenate([row, zpad]).reshape(NW, NB, B)
    col_p = jnp.concatenate([col, zpad]).reshape(NW, NB, B)

    deg_parts, dest = _sc_deg(row_p, col_p)
    deg2 = jnp.stack([deg_parts[0, :N_NODES], deg_parts[1, :N_NODES]], axis=1)

    h2 = _tc_h2(x, weight, deg2)

    zinit = jnp.zeros((TROWS, F), jnp.float32)
    acc_parts = _sc_spmm(h2, col_p, dest, zinit)

    out = _tc_out(acc_parts, h2, deg2, bias.reshape(1, F))
    return out
